# baseline (speedup 1.0000x reference)
import functools

import jax
import jax.numpy as jnp
from jax import lax
from jax.experimental import pallas as pl
from jax.experimental.pallas import tpu as pltpu
from jax.experimental.pallas import tpu_sc as plsc

NC = 2
NS = 16
NW = NC * NS

GROUP = 2
GROW = GROUP * 128

VCOLS = 7813
VPAD = VCOLS * 128


def _make_detile(V, D):
  NCOL = 4
  n_chunks = (VCOLS - 1) // NCOL
  t_all = 61
  n_loop = (t_all - 1) // 2
  mesh = plsc.VectorSubcoreMesh(
      core_axis_name="c", subcore_axis_name="s",
      num_cores=NC, num_subcores=NS)

  @functools.partial(
      pl.kernel,
      mesh=mesh,
      compiler_params=pltpu.CompilerParams(needs_layout_passes=False),
      out_type=jax.ShapeDtypeStruct((VPAD * D,), jnp.float32),
      scratch_types=(
          [pltpu.VMEM((8, 512), jnp.float32) for _ in range(8)]
          + [pltpu.VMEM((16384,), jnp.float32) for _ in range(2)]
          + [pltpu.VMEM((8, 64), jnp.float32) for _ in range(4)]
          + [pltpu.SemaphoreType.DMA for _ in range(4)]
      ),
  )
  def detile_kernel(tab_hbm, out_hbm, *scr):
    wid = lax.axis_index("s") * NC + lax.axis_index("c")
    tin = (scr[0:4], scr[4:8])
    tout = scr[8:10]
    ttail = scr[10:14]
    rsem = (scr[14], scr[15])
    wsem = (scr[16], scr[17])

    iota = lax.iota(jnp.int32, 16)
    rowv = [iota + q * 16 for q in range(8)]
    kkv = [jnp.bitwise_and(iota + d, 7) for d in range(8)]
    sbase = [(iota + q * 16) * D for q in range(8)]

    def chunk_of(t):
      return t * NW + wid

    def read_start(t, buf):
      m = chunk_of(t)
      for r in range(4):
        pltpu.async_copy(
            tab_hbm.at[pl.ds(8 * r, 8), pl.ds(m * 512, 512)],
            tin[buf][r], rsem[buf])

    def read_sync(m, buf):
      for r in range(4):
        pltpu.sync_copy(
            tab_hbm.at[pl.ds(8 * r, 8), pl.ds(m * 512, 512)],
            tin[buf][r])

    def read_wait(buf):
      for r in range(4):
        pltpu.make_async_copy(
            tab_hbm.at[pl.ds(0, 8), pl.ds(0, 512)],
            tin[buf][r], rsem[buf]).wait()

    def write_start(m, buf):
      pltpu.async_copy(
          tout[buf], out_hbm.at[pl.ds(m * 16384, 16384)], wsem[buf])

    def write_wait(buf):
      pltpu.make_async_copy(
          tout[buf], out_hbm.at[pl.ds(0, 16384)], wsem[buf]).wait()

    def transpose_chunk(buf):
      def sbody(sub, carry):
        rvq = [rowv[q] + sub * 128 for q in range(8)]
        ssq = [sbase[q] + sub * 4096 for q in range(8)]
        for r in range(4):
          for d in range(8):
            srk = kkv[d] + 8 * r
            for q in range(8):
              v = plsc.load_gather(tin[buf][r], [kkv[d], rvq[q]])
              plsc.store_scatter(tout[buf], [ssq[q] + srk], v)
        return carry
      lax.fori_loop(0, NCOL, sbody, 0)

    read_start(0, 0)
    read_start(1, 1)

    def body(u, carry):
      for half in range(2):
        buf = half
        t = u * 2 + half
        read_wait(buf)
        @pl.when(u > 0)
        def _():
          write_wait(buf)
        transpose_chunk(buf)
        write_start(chunk_of(t), buf)
        if half == 0:
          read_start(t + 2, buf)
        else:
          @pl.when(u < n_loop - 1)
          def _():
            read_start(t + 2, buf)
      return carry

    lax.fori_loop(0, n_loop, body, 0)

    read_wait(0)
    write_wait(0)
    transpose_chunk(0)
    write_start(chunk_of(t_all - 1), 0)
    write_wait(0)
    write_wait(1)

    @pl.when(wid == 0)
    def _():
      read_sync(n_chunks - 1, 0)
      transpose_chunk(0)
      pltpu.async_copy(
          tout[0], out_hbm.at[pl.ds((n_chunks - 1) * 16384, 16384)],
          wsem[0])
      write_wait(0)

    @pl.when(wid == NW - 1)
    def _():
      for r in range(4):
        pltpu.sync_copy(
            tab_hbm.at[pl.ds(8 * r, 8), pl.ds((VCOLS - 1) * 128, 64)],
            ttail[r])
      for r in range(4):
        for d in range(8):
          srk = kkv[d] + 8 * r
          for q in range(4):
            v = plsc.load_gather(ttail[r], [kkv[d], rowv[q]])
            plsc.store_scatter(tout[0], [sbase[q] + srk], v)
      pltpu.async_copy(
          tout[0].at[pl.ds(0, 2048)],
          out_hbm.at[pl.ds((VCOLS - 1) * 4096, 2048)], wsem[0])
      pltpu.make_async_copy(
          tout[0].at[pl.ds(0, 2048)],
          out_hbm.at[pl.ds(0, 2048)], wsem[0]).wait()

  return detile_kernel


def _make_gather(B, V, D):
  n_blocks = B // 128
  blocks_per_w = n_blocks // NW
  b_per_w = B // NW
  n_g = b_per_w // GROW
  SB = 8
  n_sb = blocks_per_w // SB
  mesh = plsc.VectorSubcoreMesh(
      core_axis_name="c", subcore_axis_name="s",
      num_cores=NC, num_subcores=NS)

  @functools.partial(
      pl.kernel,
      mesh=mesh,
      compiler_params=pltpu.CompilerParams(
          use_tc_tiling_on_sc=False, needs_layout_passes=False),
      out_type=jax.ShapeDtypeStruct((B * D,), jnp.float32),
      scratch_types=[
          pltpu.VMEM((b_per_w,), jnp.int32),
          pltpu.VMEM((GROW, D), jnp.float32),
          pltpu.VMEM((GROW, D), jnp.float32),
          pltpu.VMEM((SB * 4096,), jnp.float32),
          pltpu.VMEM((SB * 4096,), jnp.float32),
          pltpu.SemaphoreType.DMA,
          pltpu.SemaphoreType.DMA,
          pltpu.SemaphoreType.DMA,
          pltpu.SemaphoreType.DMA,
      ],
  )
  def gather_kernel(table_hbm, idx_hbm, out_hbm, idx_v,
                    g0, g1, o0, o1, gsem0, gsem1, wsem0, wsem1):
    wid = lax.axis_index("s") * NC + lax.axis_index("c")
    base = wid * b_per_w
    nbase = wid * blocks_per_w
    G = (g0, g1)
    O = (o0, o1)
    gsem = (gsem0, gsem1)
    wsem = (wsem0, wsem1)

    pltpu.sync_copy(idx_hbm.at[pl.ds(base, b_per_w)], idx_v)

    iota = lax.iota(jnp.int32, 16)
    rowv = [iota + q * 16 for q in range(8)]

    def gather_start(g, buf):
      pltpu.async_copy(
          table_hbm.at[idx_v.at[pl.ds(g * GROW, GROW)]], G[buf], gsem[buf])

    def gather_wait(buf):
      pltpu.make_async_copy(
          table_hbm.at[idx_v.at[pl.ds(0, GROW)]], G[buf], gsem[buf]).wait()

    def transpose_block(gbuf, ob, p):
      blk = p % 2
      rvp = [rowv[q] + blk * 128 for q in range(8)]
      sbp = [rowv[q] + p * 1024 for q in range(8)]
      def dbody(d, carry):
        t = jnp.bitwise_and(iota + d, 31)
        abase = (jnp.left_shift(jnp.right_shift(t, 3), 13)
                 + jnp.left_shift(jnp.bitwise_and(t, 7), 7))
        for q in range(8):
          v = plsc.load_gather(G[gbuf], [rvp[q], t])
          plsc.store_scatter(O[ob], [abase + sbp[q]], v)
        return carry
      lax.fori_loop(0, D, dbody, 0)

    def write_start(sb, ob):
      n0 = nbase + sb * SB
      j = n0 // 128
      c0 = lax.rem(n0, 128)
      for r in range(4):
        pltpu.async_copy(
            O[ob].at[pl.ds(r * 8192, 8192)],
            out_hbm.at[pl.ds((j * 512 + r * 128 + c0) * 1024, 8192)],
            wsem[ob])

    def write_wait(ob):
      for r in range(4):
        pltpu.make_async_copy(
            O[ob].at[pl.ds(0, 8192)],
            out_hbm.at[pl.ds(0, 8192)], wsem[ob]).wait()

    def do_sb(sb, ob):
      @pl.when(sb >= 2)
      def _():
        write_wait(ob)
      for gp in range(4):
        g = sb * 4 + gp
        gbuf = gp % 2
        gather_wait(gbuf)
        for blk in range(2):
          transpose_block(gbuf, ob, gp * 2 + blk)
        @pl.when(g + 2 < n_g)
        def _():
          gather_start(g + 2, gbuf)
      write_start(sb, ob)

    gather_start(0, 0)
    gather_start(1, 1)

    def body(u, carry):
      do_sb(u * 2, 0)
      do_sb(u * 2 + 1, 1)
      return carry

    lax.fori_loop(0, n_sb // 2, body, 0)
    do_sb(n_sb - 1, 0)
    write_wait(1)
    write_wait(0)

  return gather_kernel


@jax.jit
def kernel(x, table):
  B = x.shape[0] * x.shape[1]
  V, D = table.shape
  tab_flat = _make_detile(V, D)(table.T)
  tab_rm = tab_flat.reshape(VPAD, D)
  idx = x.T.reshape(B).astype(jnp.int32)
  out = _make_gather(B, VPAD, D)(tab_rm, idx)
  return (out.reshape(50, 4, 128, 8, 128)
             .transpose(2, 4, 0, 1, 3)
             .reshape(x.shape[0], x.shape[1], D))


# speedup vs baseline: 1.0066x; 1.0066x over previous
import functools

import jax
import jax.numpy as jnp
from jax import lax
from jax.experimental import pallas as pl
from jax.experimental.pallas import tpu as pltpu
from jax.experimental.pallas import tpu_sc as plsc

NC = 2
NS = 16
NW = NC * NS

GROUP = 2
GROW = GROUP * 128

VCOLS = 7813
VPAD = VCOLS * 128


def _make_detile(V, D):
  NCOL = 4
  n_chunks = (VCOLS - 1) // NCOL
  t_all = 61
  n_loop = (t_all - 1) // 2
  mesh = plsc.VectorSubcoreMesh(
      core_axis_name="c", subcore_axis_name="s",
      num_cores=NC, num_subcores=NS)

  @functools.partial(
      pl.kernel,
      mesh=mesh,
      compiler_params=pltpu.CompilerParams(needs_layout_passes=False),
      out_type=jax.ShapeDtypeStruct((VPAD * D,), jnp.float32),
      scratch_types=(
          [pltpu.VMEM((8, 512), jnp.float32) for _ in range(8)]
          + [pltpu.VMEM((16384,), jnp.float32) for _ in range(2)]
          + [pltpu.VMEM((8, 64), jnp.float32) for _ in range(4)]
          + [pltpu.SemaphoreType.DMA for _ in range(4)]
      ),
  )
  def detile_kernel(tab_hbm, out_hbm, *scr):
    wid = lax.axis_index("s") * NC + lax.axis_index("c")
    tin = (scr[0:4], scr[4:8])
    tout = scr[8:10]
    ttail = scr[10:14]
    rsem = (scr[14], scr[15])
    wsem = (scr[16], scr[17])

    iota = lax.iota(jnp.int32, 16)
    rowv = [iota + q * 16 for q in range(8)]
    kkv = [jnp.bitwise_and(iota + d, 7) for d in range(8)]
    sbase = [(iota + q * 16) * D for q in range(8)]

    def chunk_of(t):
      return t * NW + wid

    def read_start(t, buf):
      m = chunk_of(t)
      for r in range(4):
        pltpu.async_copy(
            tab_hbm.at[pl.ds(8 * r, 8), pl.ds(m * 512, 512)],
            tin[buf][r], rsem[buf])

    def read_sync(m, buf):
      for r in range(4):
        pltpu.sync_copy(
            tab_hbm.at[pl.ds(8 * r, 8), pl.ds(m * 512, 512)],
            tin[buf][r])

    def read_wait(buf):
      for r in range(4):
        pltpu.make_async_copy(
            tab_hbm.at[pl.ds(0, 8), pl.ds(0, 512)],
            tin[buf][r], rsem[buf]).wait()

    def write_start(m, buf):
      pltpu.async_copy(
          tout[buf], out_hbm.at[pl.ds(m * 16384, 16384)], wsem[buf])

    def write_wait(buf):
      pltpu.make_async_copy(
          tout[buf], out_hbm.at[pl.ds(0, 16384)], wsem[buf]).wait()

    def transpose_chunk(buf):
      def sbody(sub, carry):
        rvq = [rowv[q] + sub * 128 for q in range(8)]
        ssq = [sbase[q] + sub * 4096 for q in range(8)]
        for r in range(4):
          for d in range(8):
            srk = kkv[d] + 8 * r
            for q in range(8):
              v = plsc.load_gather(tin[buf][r], [kkv[d], rvq[q]])
              plsc.store_scatter(tout[buf], [ssq[q] + srk], v)
        return carry
      lax.fori_loop(0, NCOL, sbody, 0)

    read_start(0, 0)
    read_start(1, 1)

    def body(u, carry):
      for half in range(2):
        buf = half
        t = u * 2 + half
        read_wait(buf)
        @pl.when(u > 0)
        def _():
          write_wait(buf)
        transpose_chunk(buf)
        write_start(chunk_of(t), buf)
        if half == 0:
          read_start(t + 2, buf)
        else:
          @pl.when(u < n_loop - 1)
          def _():
            read_start(t + 2, buf)
      return carry

    lax.fori_loop(0, n_loop, body, 0)

    read_wait(0)
    write_wait(0)
    transpose_chunk(0)
    write_start(chunk_of(t_all - 1), 0)
    write_wait(0)
    write_wait(1)

    @pl.when(wid == 0)
    def _():
      read_sync(n_chunks - 1, 0)
      transpose_chunk(0)
      pltpu.async_copy(
          tout[0], out_hbm.at[pl.ds((n_chunks - 1) * 16384, 16384)],
          wsem[0])
      write_wait(0)

    @pl.when(wid == NW - 1)
    def _():
      for r in range(4):
        pltpu.sync_copy(
            tab_hbm.at[pl.ds(8 * r, 8), pl.ds((VCOLS - 1) * 128, 64)],
            ttail[r])
      for r in range(4):
        for d in range(8):
          srk = kkv[d] + 8 * r
          for q in range(4):
            v = plsc.load_gather(ttail[r], [kkv[d], rowv[q]])
            plsc.store_scatter(tout[0], [sbase[q] + srk], v)
      pltpu.async_copy(
          tout[0].at[pl.ds(0, 2048)],
          out_hbm.at[pl.ds((VCOLS - 1) * 4096, 2048)], wsem[0])
      pltpu.make_async_copy(
          tout[0].at[pl.ds(0, 2048)],
          out_hbm.at[pl.ds(0, 2048)], wsem[0]).wait()

  return detile_kernel


def _make_gather(B, V, D):
  n_blocks = B // 128
  blocks_per_w = n_blocks // NW
  b_per_w = B // NW
  groups_per_w = blocks_per_w // GROUP
  n_iters = groups_per_w // 2
  mesh = plsc.VectorSubcoreMesh(
      core_axis_name="c", subcore_axis_name="s",
      num_cores=NC, num_subcores=NS)

  @functools.partial(
      pl.kernel,
      mesh=mesh,
      compiler_params=pltpu.CompilerParams(
          use_tc_tiling_on_sc=False, needs_layout_passes=False),
      out_type=jax.ShapeDtypeStruct((n_blocks * 4, 8, 128), jnp.float32),
      scratch_types=[
          pltpu.VMEM((b_per_w,), jnp.int32),
          pltpu.VMEM((GROW, D), jnp.float32),
          pltpu.VMEM((GROW, D), jnp.float32),
          pltpu.VMEM((D, 128), jnp.float32),
          pltpu.VMEM((D, 128), jnp.float32),
          pltpu.VMEM((D, 128), jnp.float32),
          pltpu.VMEM((D, 128), jnp.float32),
          pltpu.SemaphoreType.DMA,
          pltpu.SemaphoreType.DMA,
          pltpu.SemaphoreType.DMA,
          pltpu.SemaphoreType.DMA,
          pltpu.SemaphoreType.DMA,
          pltpu.SemaphoreType.DMA,
      ],
  )
  def gather_kernel(table_hbm, idx_hbm, out_hbm, idx_v,
                    g0, g1, o00, o01, o10, o11,
                    gsem0, gsem1, os00, os01, os10, os11):
    wid = lax.axis_index("s") * NC + lax.axis_index("c")
    base = wid * b_per_w
    nbase = wid * blocks_per_w
    G = (g0, g1)
    O = ((o00, o01), (o10, o11))
    gsem = (gsem0, gsem1)
    osem = ((os00, os01), (os10, os11))

    pltpu.sync_copy(idx_hbm.at[pl.ds(base, b_per_w)], idx_v)

    iota = lax.iota(jnp.int32, 16)
    rowv = [iota + q * 16 for q in range(8)]

    def gather_start(g, buf):
      pltpu.async_copy(
          table_hbm.at[idx_v.at[pl.ds(g * GROW, GROW)]], G[buf], gsem[buf])

    def gather_wait(buf):
      pltpu.make_async_copy(
          table_hbm.at[idx_v.at[pl.ds(0, GROW)]], G[buf], gsem[buf]).wait()

    def out_start(n, half, blk):
      j = n // 128
      c = lax.rem(n, 128)
      for r in range(4):
        pltpu.async_copy(
            O[half][blk].at[pl.ds(r * 8, 8), :],
            out_hbm.at[j * 512 + r * 128 + c],
            osem[half][blk])

    def out_drain(half, blk):
      for r in range(4):
        pltpu.make_async_copy(
            O[half][blk].at[pl.ds(r * 8, 8), :],
            out_hbm.at[0], osem[half][blk]).wait()

    def transpose_block(buf, half, blk):
      def dbody(d, carry):
        col = jnp.bitwise_and(iota + d, 31)
        for q in range(8):
          v = plsc.load_gather(G[buf], [rowv[q] + blk * 128, col])
          plsc.store_scatter(O[half][blk], [col, rowv[q]], v)
        return carry
      lax.fori_loop(0, D, dbody, 0)

    gather_start(0, 0)

    def body(u, carry):
      for half in range(2):
        buf, nbuf = half, 1 - half
        g = u * 2 + half
        gather_wait(buf)
        if half == 0:
          gather_start(g + 1, nbuf)
        else:
          @pl.when(u < n_iters - 1)
          def _():
            gather_start(g + 1, nbuf)
        for blk in range(GROUP):
          @pl.when(u > 0)
          def _():
            out_drain(half, blk)
          transpose_block(buf, half, blk)
          out_start(nbase + g * GROUP + blk, half, blk)
      return carry

    lax.fori_loop(0, n_iters, body, 0)
    for half in range(2):
      for blk in range(GROUP):
        out_drain(half, blk)

  return gather_kernel


@jax.jit
def kernel(x, table):
  B = x.shape[0] * x.shape[1]
  V, D = table.shape
  tab_flat = _make_detile(V, D)(table.T)
  tab_rm = tab_flat.reshape(VPAD, D)
  idx = x.T.reshape(B).astype(jnp.int32)
  out = _make_gather(B, VPAD, D)(tab_rm, idx)
  return (out.reshape(50, 4, 128, 8, 128)
             .transpose(2, 4, 0, 1, 3)
             .reshape(x.shape[0], x.shape[1], D))
